# Initial kernel scaffold; baseline (speedup 1.0000x reference)
import jax
import jax.numpy as jnp
from jax.experimental import pallas as pl


def kernel(z_e, codebook):
    raise NotImplementedError("write your pallas kernel here")


# speedup vs baseline: 3.7945x; 3.7945x over previous
import functools

import jax
import jax.numpy as jnp
from jax import lax
from jax.experimental import pallas as pl
from jax.experimental.pallas import tpu as pltpu
from jax.experimental.pallas import tpu_sc as plsc

B, D, HW = 32, 32, 64 * 64
E = 512
_LOSS_SCALE = 1.25 / (B * HW * D)


def _tc_body(cb_ref, z_ref, idx_ref, loss_ref):
    b = pl.program_id(0)
    zb = z_ref[0].reshape(D, HW)
    cb = cb_ref[...]

    nscores2 = lax.dot_general(-2.0 * cb, zb, (((1,), (0,)), ((), ())),
                               preferred_element_type=jnp.float32)
    e_sq = jnp.sum(cb * cb, axis=1, keepdims=True)
    z_sq = jnp.sum(zb * zb, axis=0, keepdims=True)
    t = z_sq + e_sq
    dist = t + nscores2

    m = jnp.min(dist, axis=0, keepdims=True)
    eidx = lax.broadcasted_iota(jnp.int32, (E, 1), 0).astype(jnp.float32)
    idx_f = jnp.min(jnp.where(dist == m, eidx, float(E)), axis=0)
    idx_ref[0, 0] = idx_f.astype(jnp.int32)

    @pl.when(b == 0)
    def _init():
        loss_ref[0, 0] = 0.0

    loss_ref[0, 0] += jnp.sum(m)

    @pl.when(b == B - 1)
    def _fin():
        loss_ref[0, 0] = loss_ref[0, 0] * _LOSS_SCALE


def _tc_stage(codebook, z3):
    return pl.pallas_call(
        _tc_body,
        grid=(B,),
        in_specs=[
            pl.BlockSpec((E, D), lambda b: (0, 0)),
            pl.BlockSpec((1, D, 64, 64), lambda b: (b, 0, 0, 0)),
        ],
        out_specs=[
            pl.BlockSpec((1, 1, HW), lambda b: (b, 0, 0)),
            pl.BlockSpec(memory_space=pltpu.SMEM),
        ],
        out_shape=[
            jax.ShapeDtypeStruct((B, 1, HW), jnp.int32),
            jax.ShapeDtypeStruct((1, 1), jnp.float32),
        ],
    )(codebook, z3)


_ROWS = 16
_NCHUNK = 64 // _ROWS


def _sc_gather(cbt, idx4):
    mesh = plsc.VectorSubcoreMesh(core_axis_name="c", subcore_axis_name="s")

    @functools.partial(
        pl.kernel,
        out_type=jax.ShapeDtypeStruct((B, D, 64, 64), jnp.float32),
        mesh=mesh,
        compiler_params=pltpu.CompilerParams(
            use_tc_tiling_on_sc=False, needs_layout_passes=False),
        scratch_types=[
            pltpu.VMEM((D * E,), jnp.float32),
            pltpu.VMEM((_ROWS * 64,), jnp.int32),
            pltpu.VMEM((D, _ROWS, 64), jnp.float32),
        ],
    )
    def body(cbt_hbm, idx_hbm, zq_hbm, cbt_v, idx_v, out_v):
        w = lax.axis_index("s") * 2 + lax.axis_index("c")
        pltpu.sync_copy(cbt_hbm, cbt_v)
        for c in range(_NCHUNK):
            pltpu.sync_copy(
                idx_hbm.at[w, 0, pl.ds(c * _ROWS * 64, _ROWS * 64)], idx_v)

            def r_body(r, _):
                for q in range(4):
                    iv = idx_v[pl.ds(r * 64 + q * 16, 16)]
                    for d in range(D):
                        out_v[d, r, pl.ds(q * 16, 16)] = plsc.load_gather(
                            cbt_v, [iv + (d * E)])
                return 0

            lax.fori_loop(0, _ROWS, r_body, 0)
            pltpu.sync_copy(out_v, zq_hbm.at[w, :, pl.ds(c * _ROWS, _ROWS)])

    return body(cbt, idx4)


def kernel(z_e, codebook):
    idx3, loss = _tc_stage(codebook, z_e)
    z_q = _sc_gather(codebook.T.reshape(-1), idx3)
    indices = idx3.reshape(B, 64, 64)
    return (z_q, indices, loss.reshape(()))


# speedup vs baseline: 4.3925x; 1.1576x over previous
import functools

import jax
import jax.numpy as jnp
from jax import lax
from jax.experimental import pallas as pl
from jax.experimental.pallas import tpu as pltpu
from jax.experimental.pallas import tpu_sc as plsc

B, D, HW = 32, 32, 64 * 64
E = 512
_LOSS_SCALE = 1.25 / (B * HW * D)


def _tc_body(cb_ref, z_ref, idx_ref, loss_ref):
    b = pl.program_id(0)
    zb = z_ref[0].reshape(D, HW)
    cb = cb_ref[...]

    nscores2 = lax.dot_general(-2.0 * cb, zb, (((1,), (0,)), ((), ())),
                               preferred_element_type=jnp.float32)
    e_sq = jnp.sum(cb * cb, axis=1, keepdims=True)
    z_sq = jnp.sum(zb * zb, axis=0, keepdims=True)
    t = z_sq + e_sq
    dist = t + nscores2

    m = jnp.min(dist, axis=0, keepdims=True)
    eidx = lax.broadcasted_iota(jnp.int32, (E, 1), 0).astype(jnp.float32)
    idx_f = jnp.min(jnp.where(dist == m, eidx, float(E)), axis=0)
    idx_ref[0, 0] = idx_f.astype(jnp.int32)

    @pl.when(b == 0)
    def _init():
        loss_ref[0, 0] = 0.0

    loss_ref[0, 0] += jnp.sum(m)

    @pl.when(b == B - 1)
    def _fin():
        loss_ref[0, 0] = loss_ref[0, 0] * _LOSS_SCALE


def _tc_stage(codebook, z3):
    return pl.pallas_call(
        _tc_body,
        grid=(B,),
        in_specs=[
            pl.BlockSpec((E, D), lambda b: (0, 0)),
            pl.BlockSpec((1, D, 64, 64), lambda b: (b, 0, 0, 0)),
        ],
        out_specs=[
            pl.BlockSpec((1, 1, HW), lambda b: (b, 0, 0)),
            pl.BlockSpec(memory_space=pltpu.SMEM),
        ],
        out_shape=[
            jax.ShapeDtypeStruct((B, 1, HW), jnp.int32),
            jax.ShapeDtypeStruct((1, 1), jnp.float32),
        ],
    )(codebook, z3)


_CHUNK = 1024
_NCHUNK = HW // _CHUNK


def _sc_gather(cbt, idx3):
    mesh = plsc.VectorSubcoreMesh(core_axis_name="c", subcore_axis_name="s")

    @functools.partial(
        pl.kernel,
        out_type=jax.ShapeDtypeStruct((B, D, HW), jnp.float32),
        mesh=mesh,
        compiler_params=pltpu.CompilerParams(
            use_tc_tiling_on_sc=False, needs_layout_passes=False),
        scratch_types=[
            pltpu.VMEM((D * E,), jnp.float32),
            pltpu.VMEM((HW,), jnp.int32),
            pltpu.VMEM((D, _CHUNK), jnp.float32),
            pltpu.VMEM((D, _CHUNK), jnp.float32),
            pltpu.SemaphoreType.DMA,
            pltpu.SemaphoreType.DMA,
        ],
    )
    def body(cbt_hbm, idx_hbm, zq_hbm, cbt_v, idx_v, out0, out1, sem0, sem1):
        w = lax.axis_index("s") * 2 + lax.axis_index("c")
        bufs, sems = (out0, out1), (sem0, sem1)
        pltpu.sync_copy(cbt_hbm, cbt_v)
        pltpu.sync_copy(idx_hbm.at[w, 0], idx_v)
        for c in range(_NCHUNK):
            buf, sem = bufs[c % 2], sems[c % 2]
            if c >= 2:
                pltpu.make_async_copy(
                    buf, zq_hbm.at[w, :, pl.ds((c - 2) * _CHUNK, _CHUNK)],
                    sem).wait()

            @plsc.parallel_loop(0, _CHUNK // 16, 1, unroll=4)
            def g_body(g, _c=c, _buf=buf):
                iv = idx_v[pl.ds(_c * _CHUNK + g * 16, 16)]
                for d in range(D):
                    _buf[d, pl.ds(g * 16, 16)] = plsc.load_gather(
                        cbt_v, [iv + (d * E)])

            pltpu.async_copy(
                buf, zq_hbm.at[w, :, pl.ds(c * _CHUNK, _CHUNK)], sem)
        for c in (_NCHUNK - 2, _NCHUNK - 1):
            pltpu.make_async_copy(
                bufs[c % 2], zq_hbm.at[w, :, pl.ds(c * _CHUNK, _CHUNK)],
                sems[c % 2]).wait()

    return body(cbt, idx3)


def kernel(z_e, codebook):
    idx3, loss = _tc_stage(codebook, z_e)
    zq3 = _sc_gather(codebook.T.reshape(-1), idx3)
    z_q = zq3.reshape(B, D, 64, 64)
    indices = idx3.reshape(B, 64, 64)
    return (z_q, indices, loss.reshape(()))
